# Initial kernel scaffold; baseline (speedup 1.0000x reference)
import jax
import jax.numpy as jnp
from jax.experimental import pallas as pl


def kernel(x, edge_index, batch, W1, b1, W2, b2, W3, b3, W4, b4, W5, b5, W6, b6, Wc, bc):
    raise NotImplementedError("write your pallas kernel here")


# speedup vs baseline: 16.3551x; 16.3551x over previous
import functools

import jax
import jax.numpy as jnp
from jax import lax
from jax.experimental import pallas as pl
from jax.experimental.pallas import tpu as pltpu
from jax.experimental.pallas import tpu_sc as plsc

N = 10000
NP = 10240
E = 320000
EP = 327680
G = 128
K = 50
DC = 97
DP = 112
NC = 2
NS = 16
ACC_R = NP + 256
ZR = ACC_R // NS
CPT = EP // (NC * NS) // 128
DUMP = G * K


def _mesh():
    return plsc.VectorSubcoreMesh(
        core_axis_name="c", subcore_axis_name="s", num_cores=NC, num_subcores=NS
    )


def _sc_degree(dst2d):
    def body(dst_hbm, out0, out1, dstb, onesb, zbuf, acc, sem):
        c = lax.axis_index("c")
        s = lax.axis_index("s")
        tile = c * NS + s
        pltpu.sync_copy(dst_hbm.at[pl.ds(tile * CPT, CPT)], dstb)
        z16 = jnp.zeros((16,), jnp.float32)

        @pl.loop(0, ZR // 16)
        def _(r):
            zbuf[pl.ds(r * 16, 16)] = z16

        pltpu.sync_copy(zbuf, acc.at[pl.ds(s * ZR, ZR)])
        ones16 = jnp.ones((16,), jnp.float32)
        for i in range(8):
            onesb[pl.ds(i * 16, 16)] = ones16
        plsc.subcore_barrier()

        @pl.loop(0, CPT)
        def _(j):
            pltpu.sync_copy(onesb, acc.at[dstb.at[j]], add=True)

        plsc.subcore_barrier()

        @pl.when(c == 0)
        def _():
            pltpu.sync_copy(acc.at[pl.ds(s * 640, 640)], out0.at[pl.ds(s * 640, 640)])

        @pl.when(c == 1)
        def _():
            pltpu.sync_copy(acc.at[pl.ds(s * 640, 640)], out1.at[pl.ds(s * 640, 640)])

    f = pl.kernel(
        body,
        out_type=(
            jax.ShapeDtypeStruct((NP,), jnp.float32),
            jax.ShapeDtypeStruct((NP,), jnp.float32),
        ),
        mesh=_mesh(),
        compiler_params=pltpu.CompilerParams(use_tc_tiling_on_sc=False, needs_layout_passes=False),
        scratch_types=[
            pltpu.VMEM((CPT, 128), jnp.int32),
            pltpu.VMEM((128,), jnp.float32),
            pltpu.VMEM((ZR,), jnp.float32),
            pltpu.VMEM_SHARED((ACC_R,), jnp.float32),
            pltpu.SemaphoreType.DMA,
        ],
    )
    return f(dst2d)


def _sc_segsum(g, src2d, dst2d, F):
    gshape = (NP,) if F == 1 else (NP, F)
    oshape = gshape
    accshape = (ACC_R,) if F == 1 else (ACC_R, F)
    zshape = (ZR,) if F == 1 else (ZR, F)
    rshape = (128,) if F == 1 else (128, F)

    def body(g_hbm, src_hbm, dst_hbm, out0, out1,
             srcb, dstb, rows0, rows1, zbuf, acc, sem0, sem1):
        c = lax.axis_index("c")
        s = lax.axis_index("s")
        tile = c * NS + s
        pltpu.sync_copy(src_hbm.at[pl.ds(tile * CPT, CPT)], srcb)
        pltpu.sync_copy(dst_hbm.at[pl.ds(tile * CPT, CPT)], dstb)
        z16 = jnp.zeros((16,), jnp.float32)
        if F == 1:
            @pl.loop(0, ZR // 16)
            def _(r):
                zbuf[pl.ds(r * 16, 16)] = z16
        else:
            @pl.loop(0, ZR)
            def _(r):
                for cc in range(F // 16):
                    zbuf[r, pl.ds(cc * 16, 16)] = z16

        pltpu.sync_copy(zbuf, acc.at[pl.ds(s * ZR, ZR)])
        plsc.subcore_barrier()

        rows = (rows0, rows1)
        sems = (sem0, sem1)
        pltpu.async_copy(g_hbm.at[srcb.at[0]], rows0, sem0)
        pltpu.async_copy(g_hbm.at[srcb.at[1]], rows1, sem1)

        @pl.loop(0, CPT, step=2)
        def _(k):
            for b in range(2):
                j = k + b
                pltpu.make_async_copy(g_hbm.at[srcb.at[j]], rows[b], sems[b]).wait()
                pltpu.sync_copy(rows[b], acc.at[dstb.at[j]], add=True)

                @pl.when(j + 2 < CPT)
                def _():
                    pltpu.async_copy(g_hbm.at[srcb.at[j + 2]], rows[b], sems[b])

        plsc.subcore_barrier()

        @pl.when(c == 0)
        def _():
            pltpu.sync_copy(acc.at[pl.ds(s * 640, 640)], out0.at[pl.ds(s * 640, 640)])

        @pl.when(c == 1)
        def _():
            pltpu.sync_copy(acc.at[pl.ds(s * 640, 640)], out1.at[pl.ds(s * 640, 640)])

    f = pl.kernel(
        body,
        out_type=(
            jax.ShapeDtypeStruct(oshape, jnp.float32),
            jax.ShapeDtypeStruct(oshape, jnp.float32),
        ),
        mesh=_mesh(),
        compiler_params=pltpu.CompilerParams(use_tc_tiling_on_sc=False, needs_layout_passes=False),
        scratch_types=[
            pltpu.VMEM((CPT, 128), jnp.int32),
            pltpu.VMEM((CPT, 128), jnp.int32),
            pltpu.VMEM(rshape, jnp.float32),
            pltpu.VMEM(rshape, jnp.float32),
            pltpu.VMEM(zshape, jnp.float32),
            pltpu.VMEM_SHARED(accshape, jnp.float32),
            pltpu.SemaphoreType.DMA,
            pltpu.SemaphoreType.DMA,
        ],
    )
    return f(g, src2d, dst2d)


TA = NP + G * K + 256


def _sc_pool(tgt_all, xcp):
    n_chunks = (G * K) // 128

    def body(tgt_hbm, xcp_hbm, out, tgtb, posv, rowsb, sem):
        c = lax.axis_index("c")
        s = lax.axis_index("s")
        wid = c * NS + s
        pltpu.sync_copy(tgt_hbm, tgtb)
        iota16 = lax.iota(jnp.int32, 16)
        npvec = jnp.full((16,), NP, jnp.int32)

        @pl.loop(0, TA // 16)
        def _(i):
            t = tgtb[pl.ds(i * 16, 16)]
            val = jnp.minimum(iota16 + i * 16, npvec)
            plsc.store_scatter(
                posv,
                [lax.shift_right_logical(t, 7), lax.bitwise_and(t, 127)],
                val,
            )

        for chunk in range(n_chunks):
            @pl.when(wid == chunk % (NC * NS))
            def _():
                pltpu.async_copy(xcp_hbm.at[posv.at[chunk]], rowsb, sem).wait()
                pltpu.sync_copy(rowsb, out.at[pl.ds(chunk * 128, 128)])

    f = pl.kernel(
        body,
        out_type=jax.ShapeDtypeStruct((G * K, DP), jnp.float32),
        mesh=_mesh(),
        compiler_params=pltpu.CompilerParams(use_tc_tiling_on_sc=False, needs_layout_passes=False),
        scratch_types=[
            pltpu.VMEM((TA,), jnp.int32),
            pltpu.VMEM((52, 128), jnp.int32),
            pltpu.VMEM((128, DP), jnp.float32),
            pltpu.SemaphoreType.DMA,
        ],
    )
    return f(tgt_all, xcp)


_BR = 2048


def _tc_layer1(x, W1, p0, p1):
    def body(x_ref, w_ref, p0_ref, p1_ref, g_ref, dinv_ref):
        dv = lax.rsqrt(1.0 + p0_ref[...] + p1_ref[...])
        g_ref[...] = jnp.dot(x_ref[...], w_ref[...],
                             preferred_element_type=jnp.float32) * dv
        dinv_ref[...] = dv

    return pl.pallas_call(
        body,
        grid=(NP // _BR,),
        in_specs=[
            pl.BlockSpec((_BR, 128), lambda i: (i, 0)),
            pl.BlockSpec((128, 32), lambda i: (0, 0)),
            pl.BlockSpec((_BR, 1), lambda i: (i, 0)),
            pl.BlockSpec((_BR, 1), lambda i: (i, 0)),
        ],
        out_specs=(
            pl.BlockSpec((_BR, 32), lambda i: (i, 0)),
            pl.BlockSpec((_BR, 1), lambda i: (i, 0)),
        ),
        out_shape=(
            jax.ShapeDtypeStruct((NP, 32), jnp.float32),
            jax.ShapeDtypeStruct((NP, 1), jnp.float32),
        ),
    )(x, W1, p0, p1)


def _tc_layer(gprev, p0, p1, dinv, b, W, Fout):
    def body(gp_ref, p0_ref, p1_ref, dv_ref, b_ref, w_ref, x_ref, g_ref):
        dv = dv_ref[...]
        xin = jnp.tanh((gp_ref[...] + p0_ref[...] + p1_ref[...]) * dv + b_ref[...])
        x_ref[...] = xin
        g_ref[...] = jnp.dot(xin, w_ref[...],
                             preferred_element_type=jnp.float32) * dv

    return pl.pallas_call(
        body,
        grid=(NP // _BR,),
        in_specs=[
            pl.BlockSpec((_BR, 32), lambda i: (i, 0)),
            pl.BlockSpec((_BR, 32), lambda i: (i, 0)),
            pl.BlockSpec((_BR, 32), lambda i: (i, 0)),
            pl.BlockSpec((_BR, 1), lambda i: (i, 0)),
            pl.BlockSpec((1, 32), lambda i: (0, 0)),
            pl.BlockSpec((32, Fout), lambda i: (0, 0)),
        ],
        out_specs=(
            pl.BlockSpec((_BR, 32), lambda i: (i, 0)),
            pl.BlockSpec((_BR, Fout), lambda i: (i, 0)),
        ),
        out_shape=(
            jax.ShapeDtypeStruct((NP, 32), jnp.float32),
            jax.ShapeDtypeStruct((NP, Fout), jnp.float32),
        ),
    )(gprev, p0, p1, dinv, b, W)


def _tc_final_node(g4, p0, p1, dinv, b4, batch_col):
    def body(g_ref, p0_ref, p1_ref, dv_ref, b_ref, bat_ref, x4_ref, kv_ref):
        x4 = jnp.tanh((g_ref[...] + p0_ref[...] + p1_ref[...]) * dv_ref[...]
                      + b_ref[...])
        x4_ref[...] = x4
        kv_ref[...] = bat_ref[...].astype(jnp.float32) * 4.0 - x4

    return pl.pallas_call(
        body,
        grid=(NP // _BR,),
        in_specs=[
            pl.BlockSpec((_BR, 1), lambda i: (i, 0)),
            pl.BlockSpec((_BR, 1), lambda i: (i, 0)),
            pl.BlockSpec((_BR, 1), lambda i: (i, 0)),
            pl.BlockSpec((_BR, 1), lambda i: (i, 0)),
            pl.BlockSpec((1, 1), lambda i: (0, 0)),
            pl.BlockSpec((_BR, 1), lambda i: (i, 0)),
        ],
        out_specs=(
            pl.BlockSpec((_BR, 1), lambda i: (i, 0)),
            pl.BlockSpec((_BR, 1), lambda i: (i, 0)),
        ),
        out_shape=(
            jax.ShapeDtypeStruct((NP, 1), jnp.float32),
            jax.ShapeDtypeStruct((NP, 1), jnp.float32),
        ),
    )(g4, p0, p1, dinv, b4, batch_col)


_BI = 256
_BJ = 1024


def _tc_rank(kv_col, kv_row, batch_col):
    def body(kvc_ref, kvr_ref, bat_ref, tgt_ref):
        kvi = kvc_ref[...]
        bat = bat_ref[...]
        bthr = bat.astype(jnp.float32) * 4.0 - 1.0
        iidx = (pl.program_id(0) * _BI
                + lax.broadcasted_iota(jnp.int32, (_BI, 1), 0))
        acc = jnp.zeros((_BI, 1), jnp.int32)
        for cidx in range(NP // _BJ):
            kvj = kvr_ref[:, cidx * _BJ:(cidx + 1) * _BJ]
            jidx = cidx * _BJ + lax.broadcasted_iota(jnp.int32, (_BI, _BJ), 1)
            lt = kvj < kvi
            tie = (kvj == kvi) & (jidx < iidx)
            blt = kvj < bthr
            contrib = (lt | tie).astype(jnp.int32) - blt.astype(jnp.int32)
            acc = acc + jnp.sum(contrib, axis=1, keepdims=True)
        tgt = jnp.where((bat < G) & (acc < K), bat * K + acc,
                        DUMP + (iidx & 255))
        tgt_ref[...] = tgt

    return pl.pallas_call(
        body,
        grid=(NP // _BI,),
        in_specs=[
            pl.BlockSpec((_BI, 1), lambda i: (i, 0)),
            pl.BlockSpec((1, NP), lambda i: (0, 0)),
            pl.BlockSpec((_BI, 1), lambda i: (i, 0)),
        ],
        out_specs=pl.BlockSpec((_BI, 1), lambda i: (i, 0)),
        out_shape=jax.ShapeDtypeStruct((NP, 1), jnp.int32),
    )(kv_col, kv_row, batch_col)


def _tc_fill(batch_row):
    def body(bat_ref, fill_ref):
        gcol = lax.broadcasted_iota(jnp.int32, (G, 1), 0)
        cnt = jnp.zeros((G, 1), jnp.int32)
        for cidx in range(NP // _BJ):
            bj = bat_ref[:, cidx * _BJ:(cidx + 1) * _BJ]
            cnt = cnt + jnp.sum((bj == gcol).astype(jnp.int32), axis=1,
                                keepdims=True)
        rrow = lax.broadcasted_iota(jnp.int32, (1, K), 1)
        q = gcol * K + rrow
        fill_ref[...] = jnp.where(rrow >= cnt, q, DUMP + (q & 255))

    return pl.pallas_call(
        body,
        grid=(1,),
        in_specs=[pl.BlockSpec((1, NP), lambda i: (0, 0))],
        out_specs=pl.BlockSpec((G, K), lambda i: (0, 0)),
        out_shape=jax.ShapeDtypeStruct((G, K), jnp.int32),
    )(batch_row)


def _tc_head_a(rE, rO, W5p, b5):
    def body(re_ref, ro_ref, w_ref, b_ref, out_ref):
        he = jnp.maximum(jnp.dot(re_ref[...], w_ref[...],
                                 preferred_element_type=jnp.float32)
                         + b_ref[...], 0.0)
        ho = jnp.maximum(jnp.dot(ro_ref[...], w_ref[...],
                                 preferred_element_type=jnp.float32)
                         + b_ref[...], 0.0)
        out_ref[...] = jnp.maximum(he, ho)

    return pl.pallas_call(
        body,
        grid=(1,),
        in_specs=[
            pl.BlockSpec((3200, DP), lambda i: (0, 0)),
            pl.BlockSpec((3200, DP), lambda i: (0, 0)),
            pl.BlockSpec((DP, 16), lambda i: (0, 0)),
            pl.BlockSpec((1, 16), lambda i: (0, 0)),
        ],
        out_specs=pl.BlockSpec((3200, 16), lambda i: (0, 0)),
        out_shape=jax.ShapeDtypeStruct((3200, 16), jnp.float32),
    )(rE, rO, W5p, b5)


def _tc_head_b(win, W6r, b6):
    def body(w_ref, wt_ref, b_ref, out_ref):
        out_ref[...] = jnp.maximum(
            jnp.dot(w_ref[...], wt_ref[...],
                    preferred_element_type=jnp.float32) + b_ref[...], 0.0)

    return pl.pallas_call(
        body,
        grid=(1,),
        in_specs=[
            pl.BlockSpec((G * 21, 80), lambda i: (0, 0)),
            pl.BlockSpec((80, 32), lambda i: (0, 0)),
            pl.BlockSpec((1, 32), lambda i: (0, 0)),
        ],
        out_specs=pl.BlockSpec((G * 21, 32), lambda i: (0, 0)),
        out_shape=jax.ShapeDtypeStruct((G * 21, 32), jnp.float32),
    )(win, W6r, b6)


def _tc_head_c(h, Wc, bc):
    def body(h_ref, w_ref, b_ref, out_ref):
        z = jnp.dot(h_ref[...], w_ref[...],
                    preferred_element_type=jnp.float32) + b_ref[...]
        z = z - jnp.max(z, axis=1, keepdims=True)
        ez = jnp.exp(z)
        out_ref[...] = ez / jnp.sum(ez, axis=1, keepdims=True)

    return pl.pallas_call(
        body,
        grid=(1,),
        in_specs=[
            pl.BlockSpec((G, 672), lambda i: (0, 0)),
            pl.BlockSpec((672, 10), lambda i: (0, 0)),
            pl.BlockSpec((1, 10), lambda i: (0, 0)),
        ],
        out_specs=pl.BlockSpec((G, 10), lambda i: (0, 0)),
        out_shape=jax.ShapeDtypeStruct((G, 10), jnp.float32),
    )(h, Wc, bc)


def kernel(x, edge_index, batch, W1, b1, W2, b2, W3, b3, W4, b4, W5, b5, W6, b6, Wc, bc):
    f32, i32 = jnp.float32, jnp.int32

    xp = jnp.pad(x, ((0, NP - N), (0, 0)))
    src = edge_index[0]
    dst = edge_index[1]
    pad_e = EP - E
    src2d = jnp.concatenate([src, jnp.zeros((pad_e,), i32)]).reshape(EP // 128, 128)
    dst2d = jnp.concatenate(
        [dst, NP + (jnp.arange(pad_e, dtype=i32) % 256)]).reshape(EP // 128, 128)
    batch_p = jnp.concatenate([batch, jnp.full((NP - N,), G, i32)])
    batch_col = batch_p.reshape(NP, 1)

    d0, d1 = _sc_degree(dst2d)
    g1, dinv = _tc_layer1(xp, W1, d0.reshape(NP, 1), d1.reshape(NP, 1))

    p0, p1 = _sc_segsum(g1, src2d, dst2d, 32)
    x1, g2 = _tc_layer(g1, p0, p1, dinv, b1.reshape(1, 32), W2, 32)
    p0, p1 = _sc_segsum(g2, src2d, dst2d, 32)
    x2, g3 = _tc_layer(g2, p0, p1, dinv, b2.reshape(1, 32), W3, 32)
    p0, p1 = _sc_segsum(g3, src2d, dst2d, 32)
    x3, g4 = _tc_layer(g3, p0, p1, dinv, b3.reshape(1, 32), W4, 1)
    q0, q1 = _sc_segsum(g4.reshape(NP), src2d, dst2d, 1)
    x4, kv = _tc_final_node(g4, q0.reshape(NP, 1), q1.reshape(NP, 1), dinv,
                            b4.reshape(1, 1), batch_col)

    tgt = _tc_rank(kv, kv.reshape(1, NP), batch_col)
    fill = _tc_fill(batch_p.reshape(1, NP))
    tgt_all = jnp.concatenate([
        tgt.reshape(NP), fill.reshape(G * K),
        DUMP + (jnp.arange(256, dtype=i32) % 256),
    ])
    xc = jnp.concatenate([x1, x2, x3, x4], axis=1)
    xcp = jnp.pad(xc, ((0, 8), (0, DP - DC)))
    rows = _sc_pool(tgt_all, xcp)

    r3 = rows.reshape(G, K, DP)
    rE = r3[:, 0::2, :].reshape(G * 25, DP)
    rO = r3[:, 1::2, :].reshape(G * 25, DP)
    W5p = jnp.pad(W5[:, 0, :].T, ((0, DP - DC), (0, 0)))
    pooled = _tc_head_a(rE, rO, W5p, b5.reshape(1, 16)).reshape(G, 25, 16)
    win = jnp.stack([pooled[:, k:k + 21, :] for k in range(5)], axis=2)
    win = win.reshape(G * 21, 80)
    W6r = W6.transpose(2, 1, 0).reshape(80, 32)
    h2 = _tc_head_b(win, W6r, b6.reshape(1, 32))
    h2 = h2.reshape(G, 21, 32).transpose(0, 2, 1).reshape(G, 672)
    return _tc_head_c(h2, Wc, bc.reshape(1, 10))


# speedup vs baseline: 26.0959x; 1.5956x over previous
import functools

import jax
import jax.numpy as jnp
from jax import lax
from jax.experimental import pallas as pl
from jax.experimental.pallas import tpu as pltpu
from jax.experimental.pallas import tpu_sc as plsc

N = 10000
NP = 10240
E = 320000
EP = 327680
G = 128
K = 50
DC = 97
DP = 112
NC = 2
NS = 16
ACC_R = NP + 256
ZR = ACC_R // NS
CPT = EP // (NC * NS) // 128
DUMP = G * K


def _mesh():
    return plsc.VectorSubcoreMesh(
        core_axis_name="c", subcore_axis_name="s", num_cores=NC, num_subcores=NS
    )


def _sc_degree(dst2d):
    def body(dst_hbm, out0, out1, dstb, onesb, zbuf, acc, sem):
        c = lax.axis_index("c")
        s = lax.axis_index("s")
        tile = c * NS + s
        pltpu.sync_copy(dst_hbm.at[pl.ds(tile * CPT, CPT)], dstb)
        z16 = jnp.zeros((16,), jnp.float32)

        @pl.loop(0, ZR // 16)
        def _(r):
            zbuf[pl.ds(r * 16, 16)] = z16

        pltpu.sync_copy(zbuf, acc.at[pl.ds(s * ZR, ZR)])
        ones16 = jnp.ones((16,), jnp.float32)
        for i in range(8):
            onesb[pl.ds(i * 16, 16)] = ones16
        plsc.subcore_barrier()

        @pl.loop(0, CPT)
        def _(j):
            pltpu.sync_copy(onesb, acc.at[dstb.at[j]], add=True)

        plsc.subcore_barrier()

        @pl.when(c == 0)
        def _():
            pltpu.sync_copy(acc.at[pl.ds(s * 640, 640)], out0.at[pl.ds(s * 640, 640)])

        @pl.when(c == 1)
        def _():
            pltpu.sync_copy(acc.at[pl.ds(s * 640, 640)], out1.at[pl.ds(s * 640, 640)])

    f = pl.kernel(
        body,
        out_type=(
            jax.ShapeDtypeStruct((NP,), jnp.float32),
            jax.ShapeDtypeStruct((NP,), jnp.float32),
        ),
        mesh=_mesh(),
        compiler_params=pltpu.CompilerParams(use_tc_tiling_on_sc=False, needs_layout_passes=False),
        scratch_types=[
            pltpu.VMEM((CPT, 128), jnp.int32),
            pltpu.VMEM((128,), jnp.float32),
            pltpu.VMEM((ZR,), jnp.float32),
            pltpu.VMEM_SHARED((ACC_R,), jnp.float32),
            pltpu.SemaphoreType.DMA,
        ],
    )
    return f(dst2d)


def _sc_segsum(g, src2d, dst2d, F):
    gshape = (NP,) if F == 1 else (NP, F)
    oshape = gshape
    accshape = (ACC_R,) if F == 1 else (ACC_R, F)
    zshape = (ZR,) if F == 1 else (ZR, F)
    rshape = (128,) if F == 1 else (128, F)

    def body(g_hbm, src_hbm, dst_hbm, out0, out1,
             srcb, dstb, rows0, rows1, zbuf, spg, acc, sem0, sem1):
        c = lax.axis_index("c")
        s = lax.axis_index("s")
        tile = c * NS + s
        pltpu.sync_copy(src_hbm.at[pl.ds(tile * CPT, CPT)], srcb)
        pltpu.sync_copy(dst_hbm.at[pl.ds(tile * CPT, CPT)], dstb)
        pltpu.sync_copy(g_hbm.at[pl.ds(s * (NP // NS), NP // NS)],
                        spg.at[pl.ds(s * (NP // NS), NP // NS)])
        z16 = jnp.zeros((16,), jnp.float32)
        if F == 1:
            @pl.loop(0, ZR // 16)
            def _(r):
                zbuf[pl.ds(r * 16, 16)] = z16
        else:
            @pl.loop(0, ZR)
            def _(r):
                for cc in range(F // 16):
                    zbuf[r, pl.ds(cc * 16, 16)] = z16

        pltpu.sync_copy(zbuf, acc.at[pl.ds(s * ZR, ZR)])
        plsc.subcore_barrier()

        rows = (rows0, rows1)
        sems = (sem0, sem1)
        pltpu.async_copy(spg.at[srcb.at[0]], rows0, sem0)
        pltpu.async_copy(spg.at[srcb.at[1]], rows1, sem1)

        @pl.loop(0, CPT, step=2)
        def _(k):
            for b in range(2):
                j = k + b
                pltpu.make_async_copy(spg.at[srcb.at[j]], rows[b], sems[b]).wait()
                pltpu.sync_copy(rows[b], acc.at[dstb.at[j]], add=True)

                @pl.when(j + 2 < CPT)
                def _():
                    pltpu.async_copy(spg.at[srcb.at[j + 2]], rows[b], sems[b])

        plsc.subcore_barrier()

        @pl.when(c == 0)
        def _():
            pltpu.sync_copy(acc.at[pl.ds(s * 640, 640)], out0.at[pl.ds(s * 640, 640)])

        @pl.when(c == 1)
        def _():
            pltpu.sync_copy(acc.at[pl.ds(s * 640, 640)], out1.at[pl.ds(s * 640, 640)])

    f = pl.kernel(
        body,
        out_type=(
            jax.ShapeDtypeStruct(oshape, jnp.float32),
            jax.ShapeDtypeStruct(oshape, jnp.float32),
        ),
        mesh=_mesh(),
        compiler_params=pltpu.CompilerParams(use_tc_tiling_on_sc=False, needs_layout_passes=False),
        scratch_types=[
            pltpu.VMEM((CPT, 128), jnp.int32),
            pltpu.VMEM((CPT, 128), jnp.int32),
            pltpu.VMEM(rshape, jnp.float32),
            pltpu.VMEM(rshape, jnp.float32),
            pltpu.VMEM(zshape, jnp.float32),
            pltpu.VMEM_SHARED(gshape, jnp.float32),
            pltpu.VMEM_SHARED(accshape, jnp.float32),
            pltpu.SemaphoreType.DMA,
            pltpu.SemaphoreType.DMA,
        ],
    )
    return f(g, src2d, dst2d)


TA = NP + G * K + 256


def _sc_pool(tgt_all, xcp):
    n_chunks = (G * K) // 128

    def body(tgt_hbm, xcp_hbm, out, tgtb, posv, rowsb, sem):
        c = lax.axis_index("c")
        s = lax.axis_index("s")
        wid = c * NS + s
        pltpu.sync_copy(tgt_hbm, tgtb)
        iota16 = lax.iota(jnp.int32, 16)
        npvec = jnp.full((16,), NP, jnp.int32)

        @pl.loop(0, TA // 16)
        def _(i):
            t = tgtb[pl.ds(i * 16, 16)]
            val = jnp.minimum(iota16 + i * 16, npvec)
            plsc.store_scatter(
                posv,
                [lax.shift_right_logical(t, 7), lax.bitwise_and(t, 127)],
                val,
            )

        for chunk in range(n_chunks):
            @pl.when(wid == chunk % (NC * NS))
            def _():
                pltpu.async_copy(xcp_hbm.at[posv.at[chunk]], rowsb, sem).wait()
                pltpu.sync_copy(rowsb, out.at[pl.ds(chunk * 128, 128)])

    f = pl.kernel(
        body,
        out_type=jax.ShapeDtypeStruct((G * K, DP), jnp.float32),
        mesh=_mesh(),
        compiler_params=pltpu.CompilerParams(use_tc_tiling_on_sc=False, needs_layout_passes=False),
        scratch_types=[
            pltpu.VMEM((TA,), jnp.int32),
            pltpu.VMEM((52, 128), jnp.int32),
            pltpu.VMEM((128, DP), jnp.float32),
            pltpu.SemaphoreType.DMA,
        ],
    )
    return f(tgt_all, xcp)


_BR = 2048


def _tc_layer1(x, W1, p0, p1):
    def body(x_ref, w_ref, p0_ref, p1_ref, g_ref, dinv_ref):
        dv = lax.rsqrt(1.0 + p0_ref[...] + p1_ref[...])
        g_ref[...] = jnp.dot(x_ref[...], w_ref[...],
                             preferred_element_type=jnp.float32) * dv
        dinv_ref[...] = dv

    return pl.pallas_call(
        body,
        grid=(NP // _BR,),
        in_specs=[
            pl.BlockSpec((_BR, 128), lambda i: (i, 0)),
            pl.BlockSpec((128, 32), lambda i: (0, 0)),
            pl.BlockSpec((_BR, 1), lambda i: (i, 0)),
            pl.BlockSpec((_BR, 1), lambda i: (i, 0)),
        ],
        out_specs=(
            pl.BlockSpec((_BR, 32), lambda i: (i, 0)),
            pl.BlockSpec((_BR, 1), lambda i: (i, 0)),
        ),
        out_shape=(
            jax.ShapeDtypeStruct((NP, 32), jnp.float32),
            jax.ShapeDtypeStruct((NP, 1), jnp.float32),
        ),
    )(x, W1, p0, p1)


def _tc_layer(gprev, p0, p1, dinv, b, W, Fout):
    def body(gp_ref, p0_ref, p1_ref, dv_ref, b_ref, w_ref, x_ref, g_ref):
        dv = dv_ref[...]
        xin = jnp.tanh((gp_ref[...] + p0_ref[...] + p1_ref[...]) * dv + b_ref[...])
        x_ref[...] = xin
        g_ref[...] = jnp.dot(xin, w_ref[...],
                             preferred_element_type=jnp.float32) * dv

    return pl.pallas_call(
        body,
        grid=(NP // _BR,),
        in_specs=[
            pl.BlockSpec((_BR, 32), lambda i: (i, 0)),
            pl.BlockSpec((_BR, 32), lambda i: (i, 0)),
            pl.BlockSpec((_BR, 32), lambda i: (i, 0)),
            pl.BlockSpec((_BR, 1), lambda i: (i, 0)),
            pl.BlockSpec((1, 32), lambda i: (0, 0)),
            pl.BlockSpec((32, Fout), lambda i: (0, 0)),
        ],
        out_specs=(
            pl.BlockSpec((_BR, 32), lambda i: (i, 0)),
            pl.BlockSpec((_BR, Fout), lambda i: (i, 0)),
        ),
        out_shape=(
            jax.ShapeDtypeStruct((NP, 32), jnp.float32),
            jax.ShapeDtypeStruct((NP, Fout), jnp.float32),
        ),
    )(gprev, p0, p1, dinv, b, W)


def _tc_final_node(g4, p0, p1, dinv, b4, batch_col):
    def body(g_ref, p0_ref, p1_ref, dv_ref, b_ref, bat_ref, x4_ref, kv_ref):
        x4 = jnp.tanh((g_ref[...] + p0_ref[...] + p1_ref[...]) * dv_ref[...]
                      + b_ref[...])
        x4_ref[...] = x4
        kv_ref[...] = bat_ref[...].astype(jnp.float32) * 4.0 - x4

    return pl.pallas_call(
        body,
        grid=(NP // _BR,),
        in_specs=[
            pl.BlockSpec((_BR, 1), lambda i: (i, 0)),
            pl.BlockSpec((_BR, 1), lambda i: (i, 0)),
            pl.BlockSpec((_BR, 1), lambda i: (i, 0)),
            pl.BlockSpec((_BR, 1), lambda i: (i, 0)),
            pl.BlockSpec((1, 1), lambda i: (0, 0)),
            pl.BlockSpec((_BR, 1), lambda i: (i, 0)),
        ],
        out_specs=(
            pl.BlockSpec((_BR, 1), lambda i: (i, 0)),
            pl.BlockSpec((_BR, 1), lambda i: (i, 0)),
        ),
        out_shape=(
            jax.ShapeDtypeStruct((NP, 1), jnp.float32),
            jax.ShapeDtypeStruct((NP, 1), jnp.float32),
        ),
    )(g4, p0, p1, dinv, b4, batch_col)


_BI = 256
_BJ = 1024


def _tc_rank(kv_col, kv_row, batch_col):
    def body(kvc_ref, kvr_ref, bat_ref, tgt_ref):
        kvi = kvc_ref[...]
        bat = bat_ref[...]
        bthr = bat.astype(jnp.float32) * 4.0 - 1.0
        iidx = (pl.program_id(0) * _BI
                + lax.broadcasted_iota(jnp.int32, (_BI, 1), 0))
        acc = jnp.zeros((_BI, 1), jnp.int32)
        for cidx in range(NP // _BJ):
            kvj = kvr_ref[:, cidx * _BJ:(cidx + 1) * _BJ]
            jidx = cidx * _BJ + lax.broadcasted_iota(jnp.int32, (_BI, _BJ), 1)
            lt = kvj < kvi
            tie = (kvj == kvi) & (jidx < iidx)
            blt = kvj < bthr
            contrib = (lt | tie).astype(jnp.int32) - blt.astype(jnp.int32)
            acc = acc + jnp.sum(contrib, axis=1, keepdims=True)
        tgt = jnp.where((bat < G) & (acc < K), bat * K + acc,
                        DUMP + (iidx & 255))
        tgt_ref[...] = tgt

    return pl.pallas_call(
        body,
        grid=(NP // _BI,),
        in_specs=[
            pl.BlockSpec((_BI, 1), lambda i: (i, 0)),
            pl.BlockSpec((1, NP), lambda i: (0, 0)),
            pl.BlockSpec((_BI, 1), lambda i: (i, 0)),
        ],
        out_specs=pl.BlockSpec((_BI, 1), lambda i: (i, 0)),
        out_shape=jax.ShapeDtypeStruct((NP, 1), jnp.int32),
    )(kv_col, kv_row, batch_col)


def _tc_fill(batch_row):
    def body(bat_ref, fill_ref):
        gcol = lax.broadcasted_iota(jnp.int32, (G, 1), 0)
        cnt = jnp.zeros((G, 1), jnp.int32)
        for cidx in range(NP // _BJ):
            bj = bat_ref[:, cidx * _BJ:(cidx + 1) * _BJ]
            cnt = cnt + jnp.sum((bj == gcol).astype(jnp.int32), axis=1,
                                keepdims=True)
        rrow = lax.broadcasted_iota(jnp.int32, (1, K), 1)
        q = gcol * K + rrow
        fill_ref[...] = jnp.where(rrow >= cnt, q, DUMP + (q & 255))

    return pl.pallas_call(
        body,
        grid=(1,),
        in_specs=[pl.BlockSpec((1, NP), lambda i: (0, 0))],
        out_specs=pl.BlockSpec((G, K), lambda i: (0, 0)),
        out_shape=jax.ShapeDtypeStruct((G, K), jnp.int32),
    )(batch_row)


def _tc_head_a(rE, rO, W5p, b5):
    def body(re_ref, ro_ref, w_ref, b_ref, out_ref):
        he = jnp.maximum(jnp.dot(re_ref[...], w_ref[...],
                                 preferred_element_type=jnp.float32)
                         + b_ref[...], 0.0)
        ho = jnp.maximum(jnp.dot(ro_ref[...], w_ref[...],
                                 preferred_element_type=jnp.float32)
                         + b_ref[...], 0.0)
        out_ref[...] = jnp.maximum(he, ho)

    return pl.pallas_call(
        body,
        grid=(1,),
        in_specs=[
            pl.BlockSpec((3200, DP), lambda i: (0, 0)),
            pl.BlockSpec((3200, DP), lambda i: (0, 0)),
            pl.BlockSpec((DP, 16), lambda i: (0, 0)),
            pl.BlockSpec((1, 16), lambda i: (0, 0)),
        ],
        out_specs=pl.BlockSpec((3200, 16), lambda i: (0, 0)),
        out_shape=jax.ShapeDtypeStruct((3200, 16), jnp.float32),
    )(rE, rO, W5p, b5)


def _tc_head_b(win, W6r, b6):
    def body(w_ref, wt_ref, b_ref, out_ref):
        out_ref[...] = jnp.maximum(
            jnp.dot(w_ref[...], wt_ref[...],
                    preferred_element_type=jnp.float32) + b_ref[...], 0.0)

    return pl.pallas_call(
        body,
        grid=(1,),
        in_specs=[
            pl.BlockSpec((G * 21, 80), lambda i: (0, 0)),
            pl.BlockSpec((80, 32), lambda i: (0, 0)),
            pl.BlockSpec((1, 32), lambda i: (0, 0)),
        ],
        out_specs=pl.BlockSpec((G * 21, 32), lambda i: (0, 0)),
        out_shape=jax.ShapeDtypeStruct((G * 21, 32), jnp.float32),
    )(win, W6r, b6)


def _tc_head_c(h, Wc, bc):
    def body(h_ref, w_ref, b_ref, out_ref):
        z = jnp.dot(h_ref[...], w_ref[...],
                    preferred_element_type=jnp.float32) + b_ref[...]
        z = z - jnp.max(z, axis=1, keepdims=True)
        ez = jnp.exp(z)
        out_ref[...] = ez / jnp.sum(ez, axis=1, keepdims=True)

    return pl.pallas_call(
        body,
        grid=(1,),
        in_specs=[
            pl.BlockSpec((G, 672), lambda i: (0, 0)),
            pl.BlockSpec((672, 10), lambda i: (0, 0)),
            pl.BlockSpec((1, 10), lambda i: (0, 0)),
        ],
        out_specs=pl.BlockSpec((G, 10), lambda i: (0, 0)),
        out_shape=jax.ShapeDtypeStruct((G, 10), jnp.float32),
    )(h, Wc, bc)


def kernel(x, edge_index, batch, W1, b1, W2, b2, W3, b3, W4, b4, W5, b5, W6, b6, Wc, bc):
    f32, i32 = jnp.float32, jnp.int32

    xp = jnp.pad(x, ((0, NP - N), (0, 0)))
    src = edge_index[0]
    dst = edge_index[1]
    pad_e = EP - E
    src2d = jnp.concatenate([src, jnp.zeros((pad_e,), i32)]).reshape(EP // 128, 128)
    dst2d = jnp.concatenate(
        [dst, NP + (jnp.arange(pad_e, dtype=i32) % 256)]).reshape(EP // 128, 128)
    batch_p = jnp.concatenate([batch, jnp.full((NP - N,), G, i32)])
    batch_col = batch_p.reshape(NP, 1)

    d0, d1 = _sc_degree(dst2d)
    g1, dinv = _tc_layer1(xp, W1, d0.reshape(NP, 1), d1.reshape(NP, 1))

    p0, p1 = _sc_segsum(g1, src2d, dst2d, 32)
    x1, g2 = _tc_layer(g1, p0, p1, dinv, b1.reshape(1, 32), W2, 32)
    p0, p1 = _sc_segsum(g2, src2d, dst2d, 32)
    x2, g3 = _tc_layer(g2, p0, p1, dinv, b2.reshape(1, 32), W3, 32)
    p0, p1 = _sc_segsum(g3, src2d, dst2d, 32)
    x3, g4 = _tc_layer(g3, p0, p1, dinv, b3.reshape(1, 32), W4, 1)
    q0, q1 = _sc_segsum(g4.reshape(NP), src2d, dst2d, 1)
    x4, kv = _tc_final_node(g4, q0.reshape(NP, 1), q1.reshape(NP, 1), dinv,
                            b4.reshape(1, 1), batch_col)

    tgt = _tc_rank(kv, kv.reshape(1, NP), batch_col)
    fill = _tc_fill(batch_p.reshape(1, NP))
    tgt_all = jnp.concatenate([
        tgt.reshape(NP), fill.reshape(G * K),
        DUMP + (jnp.arange(256, dtype=i32) % 256),
    ])
    xc = jnp.concatenate([x1, x2, x3, x4], axis=1)
    xcp = jnp.pad(xc, ((0, 8), (0, DP - DC)))
    rows = _sc_pool(tgt_all, xcp)

    r3 = rows.reshape(G, K, DP)
    rE = r3[:, 0::2, :].reshape(G * 25, DP)
    rO = r3[:, 1::2, :].reshape(G * 25, DP)
    W5p = jnp.pad(W5[:, 0, :].T, ((0, DP - DC), (0, 0)))
    pooled = _tc_head_a(rE, rO, W5p, b5.reshape(1, 16)).reshape(G, 25, 16)
    win = jnp.stack([pooled[:, k:k + 21, :] for k in range(5)], axis=2)
    win = win.reshape(G * 21, 80)
    W6r = W6.transpose(2, 1, 0).reshape(80, 32)
    h2 = _tc_head_b(win, W6r, b6.reshape(1, 32))
    h2 = h2.reshape(G, 21, 32).transpose(0, 2, 1).reshape(G, 672)
    return _tc_head_c(h2, Wc, bc.reshape(1, 10))
